# baseline (speedup 1.0000x reference)
import functools

import jax
import jax.numpy as jnp
from jax import lax
from jax.experimental import pallas as pl
from jax.experimental.pallas import tpu as pltpu
from jax.experimental.pallas import tpu_sc as plsc

_NBUF = 4


def _gather_pool_sc(feat_t, table, batch, seq, emb):
    info = plsc.get_sparse_core_info()
    nc, ns = info.num_cores, info.num_subcores
    nw = nc * ns
    rows_w = batch // nw
    mesh = plsc.VectorSubcoreMesh(core_axis_name="c", subcore_axis_name="s")
    n_main = (seq - _NBUF) // _NBUF
    tail = _NBUF + (seq - _NBUF) % _NBUF

    @functools.partial(
        pl.kernel,
        out_type=jax.ShapeDtypeStruct((batch, emb), jnp.float32),
        mesh=mesh,
        scratch_types=(
            [pltpu.VMEM((rows_w,), jnp.int32) for _ in range(_NBUF)]
            + [pltpu.VMEM((rows_w, emb), jnp.float32) for _ in range(_NBUF)]
            + [pltpu.VMEM((rows_w, emb), jnp.float32)]
            + [pltpu.SemaphoreType.DMA for _ in range(2 * _NBUF)]
        ),
        compiler_params=pltpu.CompilerParams(use_tc_tiling_on_sc=False),
    )
    def k(feat_hbm, table_hbm, out_hbm, *refs):
        idx_bufs = refs[:_NBUF]
        row_bufs = refs[_NBUF:2 * _NBUF]
        pooled = refs[2 * _NBUF]
        sem_i = refs[2 * _NBUF + 1:3 * _NBUF + 1]
        sem_r = refs[3 * _NBUF + 1:]
        wid = lax.axis_index("s") * nc + lax.axis_index("c")
        col = wid * rows_w

        def idx_dma(t, b):
            return pltpu.make_async_copy(
                feat_hbm.at[t, pl.ds(col, rows_w)], idx_bufs[b], sem_i[b])

        def gather(b):
            return pltpu.make_async_copy(
                table_hbm.at[idx_bufs[b]], row_bufs[b], sem_r[b])

        def reduce_chunk(b):
            rows = row_bufs[b]

            def body(jj, _):
                i = jj * 4
                for u in range(4):
                    plsc.addupdate(pooled.at[i + u, pl.ds(0, 16)],
                                   rows[i + u, pl.ds(0, 16)])
                    plsc.addupdate(pooled.at[i + u, pl.ds(16, 16)],
                                   rows[i + u, pl.ds(16, 16)])
                return 0

            lax.fori_loop(0, rows_w // 4, body, 0)

        z = jnp.zeros((16,), jnp.float32)

        def zbody(i, _):
            pooled[i, pl.ds(0, 16)] = z
            pooled[i, pl.ds(16, 16)] = z
            return 0

        lax.fori_loop(0, rows_w, zbody, 0)

        for u in range(_NBUF):
            idx_dma(u, u).start()
        for u in range(_NBUF):
            idx_dma(u, u).wait()
            gather(u).start()

        def main(q, _):
            t = q * _NBUF
            for u in range(_NBUF):
                gather(u).wait()
                idx_dma(t + u + _NBUF, u).start()
                reduce_chunk(u)
                idx_dma(t + u + _NBUF, u).wait()
                gather(u).start()
            return 0

        lax.fori_loop(0, n_main, main, 0)

        for v in range(tail):
            u = (n_main * _NBUF + v) % _NBUF
            gather(u).wait()
            reduce_chunk(u)

        pltpu.sync_copy(pooled, out_hbm.at[pl.ds(col, rows_w)])

    return k(feat_t, table)


def _mlp_tc(pooled, W1, b1, W2, b2, inv_seq):
    batch = pooled.shape[0]

    def body(p_ref, w1_ref, b1_ref, w2_ref, b2_ref, o_ref):
        x = p_ref[...] * inv_seq
        h = jnp.dot(x, w1_ref[...], precision=lax.Precision.HIGHEST)
        h = jnp.maximum(h + b1_ref[...], 0.0)
        o = jnp.dot(h, w2_ref[...], precision=lax.Precision.HIGHEST)
        o_ref[...] = jax.nn.sigmoid(o + b2_ref[...])

    return pl.pallas_call(
        body,
        out_shape=jax.ShapeDtypeStruct((batch, W2.shape[1]), jnp.float32),
    )(pooled, W1, b1.reshape(1, -1), W2, b2.reshape(1, -1))


def kernel(features, table, W1, b1, W2, b2):
    batch, seq = features.shape
    emb = table.shape[1]
    feat_t = features.T
    pooled_sum = _gather_pool_sc(feat_t, table, batch, seq, emb)
    return _mlp_tc(pooled_sum, W1, b1, W2, b2, 1.0 / seq)


# speedup vs baseline: 1.7665x; 1.7665x over previous
import functools

import jax
import jax.numpy as jnp
from jax import lax
from jax.experimental import pallas as pl
from jax.experimental.pallas import tpu as pltpu
from jax.experimental.pallas import tpu_sc as plsc


_BLK = 32768
_QRT = _BLK // 4
_QSH = _QRT.bit_length() - 1


def _detile_tc(table_t):
    ndim, vocab = table_t.shape
    grid = (vocab + _BLK - 1) // _BLK

    def body(in_ref, out_ref):
        x = in_ref[...]
        parts = [jnp.transpose(x[:, j * _QRT:(j + 1) * _QRT])
                 for j in range(4)]
        out_ref[...] = jnp.concatenate(parts, axis=1)

    return pl.pallas_call(
        body,
        grid=(grid,),
        in_specs=[pl.BlockSpec((ndim, _BLK), lambda i: (0, i))],
        out_specs=pl.BlockSpec((_QRT, 4 * ndim), lambda i: (i, 0)),
        out_shape=jax.ShapeDtypeStruct((grid * _QRT, 4 * ndim),
                                       jnp.float32),
    )(table_t)


def _permute_idx(features):
    f = features
    return (f & ~(_BLK - 1)) | ((f & (_QRT - 1)) << 2) | ((f >> _QSH) & 3)


def _gather_pool_sc(features, table, batch, seq, emb):
    info = plsc.get_sparse_core_info()
    nc, ns = info.num_cores, info.num_subcores
    nw = nc * ns
    rows_w = batch // nw
    g_rows = 4
    n_groups = rows_w // g_rows
    mesh = plsc.VectorSubcoreMesh(core_axis_name="c", subcore_axis_name="s")

    @functools.partial(
        pl.kernel,
        out_type=jax.ShapeDtypeStruct((batch, emb), jnp.float32),
        mesh=mesh,
        scratch_types=(
            [pltpu.VMEM((seq,), jnp.int32) for _ in range(2 * g_rows)]
            + [
                pltpu.VMEM((g_rows * seq, emb), jnp.float32),
                pltpu.VMEM((g_rows * seq, emb), jnp.float32),
                pltpu.VMEM((rows_w, emb), jnp.float32),
                pltpu.SemaphoreType.DMA,
                pltpu.SemaphoreType.DMA,
                pltpu.SemaphoreType.DMA,
                pltpu.SemaphoreType.DMA,
            ]
        ),
        compiler_params=pltpu.CompilerParams(use_tc_tiling_on_sc=False),
    )
    def k(feat_hbm, table_hbm, out_hbm, i00, i01, i02, i03, i10, i11, i12,
          i13, rows0, rows1, pooled, si0, si1, sr0, sr1):
        wid = lax.axis_index("s") * nc + lax.axis_index("c")
        base = wid * rows_w
        idx_bufs = ((i00, i01, i02, i03), (i10, i11, i12, i13))
        row_bufs = (rows0, rows1)
        sem_i = (si0, si1)
        sem_r = (sr0, sr1)

        def idx_start(gi, b):
            for r in range(g_rows):
                pltpu.async_copy(feat_hbm.at[base + gi * g_rows + r],
                                 idx_bufs[b][r], sem_i[b])

        def idx_wait(gi, b):
            for r in range(g_rows):
                pltpu.make_async_copy(feat_hbm.at[base + gi * g_rows + r],
                                      idx_bufs[b][r], sem_i[b]).wait()

        def start_gathers(b):
            for r in range(g_rows):
                pltpu.async_copy(
                    table_hbm.at[idx_bufs[b][r]],
                    row_bufs[b].at[pl.ds(r * seq, seq)], sem_r[b])

        def wait_gathers(b):
            for r in range(g_rows):
                pltpu.make_async_copy(
                    table_hbm.at[idx_bufs[b][r]],
                    row_bufs[b].at[pl.ds(r * seq, seq)], sem_r[b]).wait()

        def reduce_group(gi, b):
            rows = row_bufs[b]
            for r in range(g_rows):
                roff = r * seq

                def body(jj, carry):
                    a0, a1, c0, c1 = carry
                    j = roff + jj * 8
                    for u in range(0, 8, 2):
                        a0 = a0 + rows[j + u, pl.ds(0, 16)]
                        a1 = a1 + rows[j + u, pl.ds(16, 16)]
                        c0 = c0 + rows[j + u + 1, pl.ds(0, 16)]
                        c1 = c1 + rows[j + u + 1, pl.ds(16, 16)]
                    return (a0, a1, c0, c1)

                z = jnp.zeros((16,), jnp.float32)
                a0, a1, c0, c1 = lax.fori_loop(0, seq // 8, body,
                                               (z, z, z, z))
                row = gi * g_rows + r
                pooled[row, pl.ds(0, 16)] = a0 + c0
                pooled[row, pl.ds(16, 16)] = a1 + c1

        idx_start(0, 0)
        idx_wait(0, 0)
        start_gathers(0)
        idx_start(1, 1)
        for gi in range(n_groups):
            b = gi % 2
            wait_gathers(b)
            if gi + 1 < n_groups:
                idx_wait(gi + 1, 1 - b)
                start_gathers(1 - b)
            if gi + 2 < n_groups:
                idx_start(gi + 2, b)
            reduce_group(gi, b)
        pltpu.sync_copy(pooled, out_hbm.at[pl.ds(base, rows_w)])

    return k(features, table)


def _mlp_tc(pooled, W1, b1, W2, b2, inv_seq):
    batch = pooled.shape[0]

    def body(p_ref, w1_ref, b1_ref, w2_ref, b2_ref, o_ref):
        x = p_ref[...] * inv_seq
        h = jnp.dot(x, w1_ref[...], precision=lax.Precision.HIGHEST)
        h = jnp.maximum(h + b1_ref[...], 0.0)
        o = jnp.dot(h, w2_ref[...], precision=lax.Precision.HIGHEST)
        o_ref[...] = jax.nn.sigmoid(o + b2_ref[...])

    return pl.pallas_call(
        body,
        out_shape=jax.ShapeDtypeStruct((batch, W2.shape[1]), jnp.float32),
    )(pooled, W1, b1.reshape(1, -1), W2, b2.reshape(1, -1))


def kernel(features, table, W1, b1, W2, b2):
    batch, seq = features.shape
    vocab, emb = table.shape
    detiled = _detile_tc(table.T)
    table_lin = detiled.reshape(-1, emb)
    idx = _permute_idx(features)
    pooled_sum = _gather_pool_sc(idx, table_lin, batch, seq, emb)
    return _mlp_tc(pooled_sum, W1, b1, W2, b2, 1.0 / seq)
